# Initial kernel scaffold; baseline (speedup 1.0000x reference)
import jax
import jax.numpy as jnp
from jax.experimental import pallas as pl


def kernel(adj_org, x, B, edge_index, current_epoch):
    raise NotImplementedError("write your pallas kernel here")


# speedup vs baseline: 1.0003x; 1.0003x over previous
import jax
import jax.numpy as jnp
from jax.experimental import pallas as pl
from jax.experimental.pallas import tpu as pltpu

N = 4096
D = 128
ALFA = 0.1
BETA = 0.95
EPS = 1e-6

BI = 256


def _sample_body(a_ref, b_ref, u_ref, hard_ref):
    a = a_ref[...]
    b = b_ref[...]
    u = u_ref[...]
    ep = ALFA * b + BETA * a
    ep = jnp.where(ep > 1.0, 1.0, ep)
    p = jnp.clip(ep, EPS, 1.0 - EPS)
    logits = jnp.log(p) - jnp.log1p(-p)
    noise = jnp.log(u) - jnp.log1p(-u)
    soft = jax.nn.sigmoid(logits + noise)
    hard_ref[...] = jnp.where(soft > 0.5, 1.0, 0.0)


def kernel(adj_org, x, B, edge_index, current_epoch):
    skey = jax.random.key(12345)
    u = jax.random.uniform(skey, (N, N), minval=EPS, maxval=1.0 - EPS,
                           dtype=jnp.float32)
    hard = pl.pallas_call(
        _sample_body,
        grid=(N // BI,),
        in_specs=[
            pl.BlockSpec((BI, N), lambda i: (i, 0)),
            pl.BlockSpec((BI, N), lambda i: (i, 0)),
            pl.BlockSpec((BI, N), lambda i: (i, 0)),
        ],
        out_specs=pl.BlockSpec((BI, N), lambda i: (i, 0)),
        out_shape=jax.ShapeDtypeStruct((N, N), jnp.float32),
    )(adj_org, B, u)
    adj_sampled = jnp.triu(hard, 1)
    adj_sampled = adj_sampled + adj_sampled.T
    nz = jnp.nonzero(adj_sampled > 0.5, size=adj_sampled.size, fill_value=-1)
    edge_index_sampled = jnp.stack(nz)
    check_nan = jnp.array(True)
    return (edge_index_sampled, x, adj_sampled, check_nan)
